# baseline (speedup 1.0000x reference)
import jax
import jax.numpy as jnp
from jax.experimental import pallas as pl
from jax.experimental.pallas import tpu as pltpu
from jax.experimental.pallas import tpu_sc as plsc

_NUM_CORES = 2
_NUM_SUBCORES = 16
_CHUNK = 64


def kernel(x, emb_table):
    seq_len = x.shape[1]
    dim = emb_table.shape[1]
    num_workers = _NUM_CORES * _NUM_SUBCORES
    rows_per = seq_len // num_workers
    n_chunks = rows_per // _CHUNK

    mesh = plsc.VectorSubcoreMesh(core_axis_name="c", subcore_axis_name="s")

    @pl.kernel(
        out_type=jax.ShapeDtypeStruct((1, seq_len, dim), emb_table.dtype),
        mesh=mesh,
        scratch_types=[
            pltpu.VMEM((2, _CHUNK, dim), emb_table.dtype),
            pltpu.SemaphoreType.DMA((2,)),
            pltpu.SemaphoreType.DMA((2,)),
        ],
    )
    def copy_rows(table_hbm, out_hbm, buf, in_sems, out_sems):
        core = jax.lax.axis_index("c")
        sub = jax.lax.axis_index("s")
        base = (core * _NUM_SUBCORES + sub) * rows_per

        def in_copy(i, slot):
            return pltpu.make_async_copy(
                table_hbm.at[pl.ds(base + i * _CHUNK, _CHUNK), :],
                buf.at[slot],
                in_sems.at[slot],
            )

        def out_copy(i, slot):
            return pltpu.make_async_copy(
                buf.at[slot],
                out_hbm.at[0].at[pl.ds(base + i * _CHUNK, _CHUNK), :],
                out_sems.at[slot],
            )

        in_copy(0, 0).start()
        if n_chunks > 1:
            in_copy(1, 1).start()
        for i in range(n_chunks):
            slot = i % 2
            in_copy(i, slot).wait()
            out_copy(i, slot).start()
            if i + 2 < n_chunks:
                out_copy(i, slot).wait()
                in_copy(i + 2, slot).start()
        for i in range(max(n_chunks - 2, 0), n_chunks):
            out_copy(i, i % 2).wait()

    return copy_rows(emb_table)


# speedup vs baseline: 1.0376x; 1.0376x over previous
import jax
import jax.numpy as jnp
from jax.experimental import pallas as pl
from jax.experimental.pallas import tpu as pltpu
from jax.experimental.pallas import tpu_sc as plsc

_NUM_CORES = 2
_NUM_SUBCORES = 16
_CHUNK = 128


def kernel(x, emb_table):
    seq_len = x.shape[1]
    dim = emb_table.shape[1]
    num_workers = _NUM_CORES * _NUM_SUBCORES
    rows_per = seq_len // num_workers
    n_chunks = rows_per // _CHUNK

    mesh = plsc.VectorSubcoreMesh(core_axis_name="c", subcore_axis_name="s")

    @pl.kernel(
        out_type=jax.ShapeDtypeStruct((1, seq_len, dim), emb_table.dtype),
        mesh=mesh,
        scratch_types=[
            pltpu.VMEM((2, _CHUNK, dim), emb_table.dtype),
            pltpu.SemaphoreType.DMA((2,)),
            pltpu.SemaphoreType.DMA((2,)),
        ],
    )
    def copy_rows(table_hbm, out_hbm, buf, in_sems, out_sems):
        core = jax.lax.axis_index("c")
        sub = jax.lax.axis_index("s")
        base = (core * _NUM_SUBCORES + sub) * rows_per

        def in_copy(i, slot):
            return pltpu.make_async_copy(
                table_hbm.at[pl.ds(base + i * _CHUNK, _CHUNK), :],
                buf.at[slot],
                in_sems.at[slot],
            )

        def out_copy(i, slot):
            return pltpu.make_async_copy(
                buf.at[slot],
                out_hbm.at[0].at[pl.ds(base + i * _CHUNK, _CHUNK), :],
                out_sems.at[slot],
            )

        in_copy(0, 0).start()
        if n_chunks > 1:
            in_copy(1, 1).start()
        for i in range(n_chunks):
            slot = i % 2
            in_copy(i, slot).wait()
            out_copy(i, slot).start()
            if i + 2 < n_chunks:
                out_copy(i, slot).wait()
                in_copy(i + 2, slot).start()
        for i in range(max(n_chunks - 2, 0), n_chunks):
            out_copy(i, i % 2).wait()

    return copy_rows(emb_table)
